# baseline (speedup 1.0000x reference)
import functools

import jax
import jax.numpy as jnp
from jax import lax
from jax.experimental import pallas as pl
from jax.experimental.pallas import tpu as pltpu
from jax.experimental.pallas import tpu_sc as plsc

BATCH = 128
Q = 1000
C = 80
N = Q * C
K = 300
KPAD = 320
NB1 = 1024
NB2 = 256
CAP1 = 4096
CAP2 = 512
ROWS_PER_WORKER = BATCH // 32
INT_MIN = -2147483648


def _mono_key(v):
    b = lax.bitcast_convert_type(v, jnp.int32)
    return b ^ ((b >> 31) & 0x7FFFFFFF)


def _make_sc_call():
    mesh = plsc.VectorSubcoreMesh(core_axis_name="c", subcore_axis_name="s")

    @functools.partial(
        pl.kernel,
        mesh=mesh,
        compiler_params=pltpu.CompilerParams(needs_layout_passes=False),
        out_type=[
            jax.ShapeDtypeStruct((BATCH, KPAD), jnp.int32),
            jax.ShapeDtypeStruct((BATCH, KPAD * 4), jnp.float32),
            jax.ShapeDtypeStruct((BATCH, KPAD), jnp.float32),
        ],
        scratch_types=[
            pltpu.VMEM((N // 2,), jnp.float32),
            pltpu.VMEM((N // 2,), jnp.float32),
            pltpu.SemaphoreType.DMA,
            pltpu.SemaphoreType.DMA,
            pltpu.VMEM((NB1 * 16,), jnp.int32),
            pltpu.VMEM((CAP1,), jnp.int32),
            pltpu.VMEM((CAP1,), jnp.int32),
            pltpu.VMEM((NB2 * 16,), jnp.int32),
            pltpu.VMEM((CAP2,), jnp.int32),
            pltpu.VMEM((CAP2,), jnp.int32),
            pltpu.VMEM((KPAD,), jnp.int32),
            pltpu.VMEM((Q * 4 * ROWS_PER_WORKER,), jnp.float32),
            pltpu.VMEM((16 * ROWS_PER_WORKER,), jnp.float32),
            pltpu.VMEM((KPAD,), jnp.int32),
            pltpu.VMEM((KPAD * 4,), jnp.float32),
            pltpu.VMEM((KPAD,), jnp.float32),
        ],
    )
    def sc_call(logits_hbm, boxes_hbm, scale_hbm,
                lab_hbm, box_hbm, sc_hbm,
                xb0, xb1, s0, s1, h1, ck, ci, h2, sk, si, qb, bxv, scv,
                olab, obx, osc):
        wid = lax.axis_index("s") * 2 + lax.axis_index("c")
        iota = lax.iota(jnp.int32, 16)
        ones = jnp.ones((16,), jnp.int32)
        zero16 = jnp.zeros((16,), jnp.int32)
        sentk = jnp.full((16,), INT_MIN, jnp.int32)
        qsh = iota >> 2
        c3 = iota & 3
        c3x2 = c3 ^ 2
        lo_mask = c3 < 2
        h1base = 8192 + iota
        h2base = iota
        n2 = N // 2

        def _start_row_dma(r):
            pltpu.async_copy(logits_hbm.at[2 * r], xb0, s0)
            pltpu.async_copy(logits_hbm.at[2 * r + 1], xb1, s1)

        r0 = wid * ROWS_PER_WORKER
        _start_row_dma(r0)
        pltpu.sync_copy(boxes_hbm.at[pl.ds(r0 * Q * 4, Q * 4 * ROWS_PER_WORKER)], bxv)
        pltpu.sync_copy(scale_hbm.at[pl.ds(r0 * 16, 16 * ROWS_PER_WORKER)], scv)

        def row_body(j, _):
            r = r0 + j

            @plsc.parallel_loop(0, NB1 * 16, 16, unroll=8)
            def z1(i):
                h1[pl.ds(i, 16)] = zero16

            @plsc.parallel_loop(0, NB2 * 16, 16, unroll=8)
            def z2(i):
                h2[pl.ds(i, 16)] = zero16

            pltpu.make_async_copy(logits_hbm.at[2 * r], xb0, s0).wait()

            @plsc.parallel_loop(0, n2, 16, unroll=4)
            def p1a(i):
                k = _mono_key(xb0[pl.ds(i, 16)])
                idx = ((k >> 18) & -16) + h1base
                plsc.addupdate_scatter(h1, [idx], ones)

            pltpu.make_async_copy(logits_hbm.at[2 * r + 1], xb1, s1).wait()

            @plsc.parallel_loop(0, n2, 16, unroll=4)
            def p1b(i):
                k = _mono_key(xb1[pl.ds(i, 16)])
                idx = ((k >> 18) & -16) + h1base
                plsc.addupdate_scatter(h1, [idx], ones)

            def sfxg(t, cc):
                carry, cnt, aboveg = cc
                base = (63 - t) * 256
                tot = h1[pl.ds(base, 16)]
                for l in range(1, 16):
                    tot = tot + h1[pl.ds(base + l * 16, 16)]
                incl = carry + jnp.sum(tot)
                hit = incl >= K
                first = hit & (cnt == 0)
                aboveg = jnp.where(first, carry, aboveg)
                return (incl, cnt + jnp.where(hit, 1, 0), aboveg)

            _, cntg, aboveg = lax.fori_loop(
                0, NB1 // 16, sfxg, (jnp.int32(0), jnp.int32(0), jnp.int32(0)))
            gg = cntg - 1

            btot = zero16
            for l in range(16):
                btot = btot + plsc.load_gather(h1, [gg * 256 + iota * 16 + l])
            rsuf = lax.rev(plsc.cumsum(lax.rev(btot, (0,))), (0,))
            inclb = aboveg + rsuf
            cntb = jnp.sum(jnp.where(inclb >= K, 1, 0))
            lsel = cntb - 1
            bb = gg * 16 + lsel
            above_s = jnp.sum(jnp.where(iota == lsel, inclb - btot, 0))
            t1 = (bb - 512) * (1 << 22)
            t1splat = jnp.broadcast_to(t1, (16,))

            @plsc.parallel_loop(0, n2, 16, unroll=4, carry=zero16)
            def p2a(i, off):
                k = _mono_key(xb0[pl.ds(i, 16)])
                m = k >= t1splat
                cs = plsc.cumsum(jnp.where(m, 1, 0))
                pos = jnp.maximum(jnp.minimum(off + cs - 1, CAP1 - 1), 0)
                plsc.store_scatter(ck, [pos], k, mask=m)
                plsc.store_scatter(ci, [pos], i + iota, mask=m)
                return off + plsc.all_reduce_population_count(m)

            @plsc.parallel_loop(0, n2, 16, unroll=4, carry=p2a)
            def p2b(i, off):
                k = _mono_key(xb1[pl.ds(i, 16)])
                m = k >= t1splat
                cs = plsc.cumsum(jnp.where(m, 1, 0))
                pos = jnp.maximum(jnp.minimum(off + cs - 1, CAP1 - 1), 0)
                plsc.store_scatter(ck, [pos], k, mask=m)
                plsc.store_scatter(ci, [pos], n2 + i + iota, mask=m)
                return off + plsc.all_reduce_population_count(m)

            @pl.when(j < ROWS_PER_WORKER - 1)
            def _():
                _start_row_dma(r + 1)

            n_cand = jnp.minimum(jnp.max(p2b), CAP1)
            ncand16 = ((n_cand + 15) // 16) * 16

            t1hi = t1 + (1 << 22)
            t1hisplat = jnp.broadcast_to(t1hi, (16,))

            @plsc.parallel_loop(0, ncand16, 16, unroll=2)
            def p3(i):
                k = ck[pl.ds(i, 16)]
                m = (k >= t1splat) & (k < t1hisplat) & ((i + iota) < n_cand)
                idx = ((k >> 10) & 0xFF0) + h2base
                plsc.addupdate_scatter(h2, [idx], ones, mask=m)

            def sfx2(t, cc):
                carry, cnt, aboveg2 = cc
                base = (15 - t) * 256
                tot = h2[pl.ds(base, 16)]
                for l in range(1, 16):
                    tot = tot + h2[pl.ds(base + l * 16, 16)]
                incl = carry + jnp.sum(tot)
                hit = (above_s + incl) >= K
                first = hit & (cnt == 0)
                aboveg2 = jnp.where(first, carry, aboveg2)
                return (incl, cnt + jnp.where(hit, 1, 0), aboveg2)

            _, cntg2, aboveg2 = lax.fori_loop(
                0, NB2 // 16, sfx2, (jnp.int32(0), jnp.int32(0), jnp.int32(0)))
            gg2 = cntg2 - 1
            btot2 = zero16
            for l in range(16):
                btot2 = btot2 + plsc.load_gather(
                    h2, [gg2 * 256 + iota * 16 + l])
            rsuf2 = lax.rev(plsc.cumsum(lax.rev(btot2, (0,))), (0,))
            inclb2 = above_s + aboveg2 + rsuf2
            cntb2 = jnp.sum(jnp.where(inclb2 >= K, 1, 0))
            dd2 = gg2 * 16 + (cntb2 - 1)
            t2 = t1 + dd2 * (1 << 14)
            t2splat = jnp.broadcast_to(t2, (16,))

            @plsc.parallel_loop(0, CAP2, 16, unroll=4)
            def zs(i):
                sk[pl.ds(i, 16)] = sentk
                si[pl.ds(i, 16)] = zero16

            @plsc.parallel_loop(0, ncand16, 16, unroll=2, carry=zero16)
            def p4(i, off):
                k = ck[pl.ds(i, 16)]
                m = (k >= t2splat) & ((i + iota) < n_cand)
                cs = plsc.cumsum(jnp.where(m, 1, 0))
                pos = jnp.maximum(jnp.minimum(off + cs - 1, CAP2 - 1), 0)
                plsc.store_scatter(sk, [pos], k, mask=m)
                plsc.store_scatter(si, [pos], ci[pl.ds(i, 16)], mask=m)
                return off + plsc.all_reduce_population_count(m)

            _ = p4

            for ks in [2 << s for s in range(9)]:
                jj = ks >> 1
                while jj >= 1:
                    if jj >= 16:
                        nb = jj // 16
                        lnb = nb.bit_length() - 1

                        @plsc.parallel_loop(0, CAP2 // 32, 1, unroll=4)
                        def cross(t, ks=ks, nb=nb, lnb=lnb):
                            v = ((t >> lnb) << (lnb + 1)) + (t & (nb - 1))
                            i1 = v * 16
                            i2 = (v + nb) * 16
                            ak = sk[pl.ds(i1, 16)]
                            bk = sk[pl.ds(i2, 16)]
                            ai = si[pl.ds(i1, 16)]
                            bi = si[pl.ds(i2, 16)]
                            dirn = (i1 & ks) == 0
                            cbe = (ak > bk) | ((ak == bk) & (ai < bi))
                            cond = cbe == dirn
                            sk[pl.ds(i1, 16)] = jnp.where(cond, ak, bk)
                            sk[pl.ds(i2, 16)] = jnp.where(cond, bk, ak)
                            si[pl.ds(i1, 16)] = jnp.where(cond, ai, bi)
                            si[pl.ds(i2, 16)] = jnp.where(cond, bi, ai)
                    else:
                        @plsc.parallel_loop(0, CAP2 // 16, 1, unroll=4)
                        def inner(v, ks=ks, jj=jj):
                            base = v * 16
                            ak = sk[pl.ds(base, 16)]
                            ai = si[pl.ds(base, 16)]
                            pidx = base + (iota ^ jj)
                            bk = plsc.load_gather(sk, [pidx])
                            bi = plsc.load_gather(si, [pidx])
                            dirv = ((base + iota) & ks) == 0
                            keepf = (iota & jj) == 0
                            cbe = (ak > bk) | ((ak == bk) & (ai < bi))
                            cond = (cbe == dirv) == keepf
                            sk[pl.ds(base, 16)] = jnp.where(cond, ak, bk)
                            si[pl.ds(base, 16)] = jnp.where(cond, ai, bi)
                    jj >>= 1

            @plsc.parallel_loop(0, KPAD, 16, unroll=2)
            def p5(t):
                k = sk[pl.ds(t, 16)]
                ix = si[pl.ds(t, 16)]
                v = lax.bitcast_convert_type(
                    k ^ ((k >> 31) & 0x7FFFFFFF), jnp.float32)
                sc = 1.0 / (1.0 + jnp.exp(-v))
                q = lax.div(ix, C)
                olab[pl.ds(t, 16)] = ix - q * C
                osc[pl.ds(t, 16)] = sc
                qb[pl.ds(t, 16)] = q

            scvv = scv[pl.ds(j * 16, 16)]

            boff = j * (Q * 4)

            @plsc.parallel_loop(0, KPAD * 4, 16, unroll=4)
            def p6(t):
                qv = plsc.load_gather(qb, [(t >> 2) + qsh])
                g = plsc.load_gather(bxv, [boff + qv * 4 + c3])
                p = plsc.load_gather(bxv, [boff + qv * 4 + c3x2])
                res = jnp.where(lo_mask, g - 0.5 * p, p + 0.5 * g)
                obx[pl.ds(t, 16)] = res * scvv

            pltpu.sync_copy(olab, lab_hbm.at[r])
            pltpu.sync_copy(obx, box_hbm.at[r])
            pltpu.sync_copy(osc, sc_hbm.at[r])
            return 0

        lax.fori_loop(0, ROWS_PER_WORKER, row_body, 0)

    return sc_call


_sc_call = _make_sc_call()


def kernel(pred_logits, pred_boxes, orig_target_sizes):
    logits2d = pred_logits.reshape(BATCH * 2, N // 2)
    boxes2d = pred_boxes.reshape(BATCH * Q * 4)
    scale16 = jnp.tile(orig_target_sizes, (1, 8)).reshape(-1)
    lab_p, box_p, sc_p = _sc_call(logits2d, boxes2d, scale16)
    labels = lab_p[:, :K]
    boxes = box_p.reshape(BATCH, KPAD, 4)[:, :K]
    scores = sc_p[:, :K]
    return (labels, boxes, scores)


# speedup vs baseline: 1.5236x; 1.5236x over previous
import functools

import jax
import jax.numpy as jnp
from jax import lax
from jax.experimental import pallas as pl
from jax.experimental.pallas import tpu as pltpu
from jax.experimental.pallas import tpu_sc as plsc

BATCH = 128
Q = 1000
C = 80
N = Q * C
K = 300
KPAD = 320
NB1 = 1024
NB2 = 256
CAP1 = 4096
CAP2 = 512
ROWS_PER_WORKER = BATCH // 32
INT_MIN = -2147483648


def _mono_key(v):
    b = lax.bitcast_convert_type(v, jnp.int32)
    return b ^ ((b >> 31) & 0x7FFFFFFF)


def _make_sc_call():
    mesh = plsc.VectorSubcoreMesh(core_axis_name="c", subcore_axis_name="s")

    @functools.partial(
        pl.kernel,
        mesh=mesh,
        compiler_params=pltpu.CompilerParams(needs_layout_passes=False),
        out_type=[
            jax.ShapeDtypeStruct((BATCH, KPAD), jnp.int32),
            jax.ShapeDtypeStruct((BATCH, KPAD * 4), jnp.float32),
            jax.ShapeDtypeStruct((BATCH, KPAD), jnp.float32),
        ],
        scratch_types=[
            pltpu.VMEM((N,), jnp.float32),
            pltpu.VMEM((NB1 * 16,), jnp.int32),
            pltpu.VMEM((CAP1,), jnp.int32),
            pltpu.VMEM((CAP1,), jnp.int32),
            pltpu.VMEM((NB2 * 16,), jnp.int32),
            pltpu.VMEM((CAP2,), jnp.int32),
            pltpu.VMEM((CAP2,), jnp.int32),
            pltpu.VMEM((KPAD,), jnp.int32),
            pltpu.VMEM((Q * 4,), jnp.float32),
            pltpu.VMEM((16,), jnp.float32),
            pltpu.VMEM((KPAD,), jnp.int32),
            pltpu.VMEM((KPAD * 4,), jnp.float32),
            pltpu.VMEM((KPAD,), jnp.float32),
        ],
    )
    def sc_call(logits_hbm, boxes_hbm, scale_hbm,
                lab_hbm, box_hbm, sc_hbm,
                xb, h1, ck, ci, h2, sk, si, qb, bxv, scv,
                olab, obx, osc):
        wid = lax.axis_index("s") * 2 + lax.axis_index("c")
        iota = lax.iota(jnp.int32, 16)
        ones = jnp.ones((16,), jnp.int32)
        zero16 = jnp.zeros((16,), jnp.int32)
        sentk = jnp.full((16,), INT_MIN, jnp.int32)
        qsh = iota >> 2
        c3 = iota & 3
        c3x2 = c3 ^ 2
        lo_mask = c3 < 2
        h1base = 8192 + iota
        h2base = iota

        def row_body(j, _):
            r = wid * ROWS_PER_WORKER + j
            pltpu.sync_copy(logits_hbm.at[r], xb)
            pltpu.sync_copy(boxes_hbm.at[r], bxv)
            pltpu.sync_copy(scale_hbm.at[r], scv)

            @plsc.parallel_loop(0, NB1 * 16, 16, unroll=8)
            def z1(i):
                h1[pl.ds(i, 16)] = zero16

            @plsc.parallel_loop(0, NB2 * 16, 16, unroll=8)
            def z2(i):
                h2[pl.ds(i, 16)] = zero16

            @plsc.parallel_loop(0, N, 16, unroll=4)
            def p1(i):
                k = _mono_key(xb[pl.ds(i, 16)])
                idx = ((k >> 18) & -16) + h1base
                plsc.addupdate_scatter(h1, [idx], ones)

            def sfxg(t, cc):
                carry, cnt, aboveg = cc
                base = (63 - t) * 256
                tot = h1[pl.ds(base, 16)]
                for l in range(1, 16):
                    tot = tot + h1[pl.ds(base + l * 16, 16)]
                incl = carry + jnp.sum(tot)
                hit = incl >= K
                first = hit & (cnt == 0)
                aboveg = jnp.where(first, carry, aboveg)
                return (incl, cnt + jnp.where(hit, 1, 0), aboveg)

            _, cntg, aboveg = lax.fori_loop(
                0, NB1 // 16, sfxg, (jnp.int32(0), jnp.int32(0), jnp.int32(0)))
            gg = cntg - 1

            btot = zero16
            for l in range(16):
                btot = btot + plsc.load_gather(h1, [gg * 256 + iota * 16 + l])
            rsuf = lax.rev(plsc.cumsum(lax.rev(btot, (0,))), (0,))
            inclb = aboveg + rsuf
            cntb = jnp.sum(jnp.where(inclb >= K, 1, 0))
            lsel = cntb - 1
            bb = gg * 16 + lsel
            above_s = jnp.sum(jnp.where(iota == lsel, inclb - btot, 0))
            t1 = (bb - 512) * (1 << 22)
            t1splat = jnp.broadcast_to(t1, (16,))

            @plsc.parallel_loop(0, N, 16, unroll=4, carry=zero16)
            def p2(i, off):
                k = _mono_key(xb[pl.ds(i, 16)])
                m = k >= t1splat
                cs = plsc.cumsum(jnp.where(m, 1, 0))
                pos = jnp.maximum(jnp.minimum(off + cs - 1, CAP1 - 1), 0)
                plsc.store_scatter(ck, [pos], k, mask=m)
                plsc.store_scatter(ci, [pos], i + iota, mask=m)
                return off + plsc.all_reduce_population_count(m)

            n_cand = jnp.minimum(jnp.max(p2), CAP1)
            ncand16 = ((n_cand + 15) // 16) * 16

            t1hi = t1 + (1 << 22)
            t1hisplat = jnp.broadcast_to(t1hi, (16,))

            @plsc.parallel_loop(0, ncand16, 16, unroll=2)
            def p3(i):
                k = ck[pl.ds(i, 16)]
                m = (k >= t1splat) & (k < t1hisplat) & ((i + iota) < n_cand)
                idx = ((k >> 10) & 0xFF0) + h2base
                plsc.addupdate_scatter(h2, [idx], ones, mask=m)

            def sfx2(t, cc):
                carry, cnt, aboveg2 = cc
                base = (15 - t) * 256
                tot = h2[pl.ds(base, 16)]
                for l in range(1, 16):
                    tot = tot + h2[pl.ds(base + l * 16, 16)]
                incl = carry + jnp.sum(tot)
                hit = (above_s + incl) >= K
                first = hit & (cnt == 0)
                aboveg2 = jnp.where(first, carry, aboveg2)
                return (incl, cnt + jnp.where(hit, 1, 0), aboveg2)

            _, cntg2, aboveg2 = lax.fori_loop(
                0, NB2 // 16, sfx2, (jnp.int32(0), jnp.int32(0), jnp.int32(0)))
            gg2 = cntg2 - 1
            btot2 = zero16
            for l in range(16):
                btot2 = btot2 + plsc.load_gather(
                    h2, [gg2 * 256 + iota * 16 + l])
            rsuf2 = lax.rev(plsc.cumsum(lax.rev(btot2, (0,))), (0,))
            inclb2 = above_s + aboveg2 + rsuf2
            cntb2 = jnp.sum(jnp.where(inclb2 >= K, 1, 0))
            dd2 = gg2 * 16 + (cntb2 - 1)
            t2 = t1 + dd2 * (1 << 14)
            t2splat = jnp.broadcast_to(t2, (16,))

            @plsc.parallel_loop(0, CAP2, 16, unroll=4)
            def zs(i):
                sk[pl.ds(i, 16)] = sentk
                si[pl.ds(i, 16)] = zero16

            @plsc.parallel_loop(0, ncand16, 16, unroll=2, carry=zero16)
            def p4(i, off):
                k = ck[pl.ds(i, 16)]
                m = (k >= t2splat) & ((i + iota) < n_cand)
                cs = plsc.cumsum(jnp.where(m, 1, 0))
                pos = jnp.maximum(jnp.minimum(off + cs - 1, CAP2 - 1), 0)
                plsc.store_scatter(sk, [pos], k, mask=m)
                plsc.store_scatter(si, [pos], ci[pl.ds(i, 16)], mask=m)
                return off + plsc.all_reduce_population_count(m)

            _ = p4

            for ks in [2 << s for s in range(9)]:
                jj = ks >> 1
                while jj >= 1:
                    if jj >= 16:
                        nb = jj // 16
                        lnb = nb.bit_length() - 1

                        @plsc.parallel_loop(0, CAP2 // 32, 1, unroll=4)
                        def cross(t, ks=ks, nb=nb, lnb=lnb):
                            v = ((t >> lnb) << (lnb + 1)) + (t & (nb - 1))
                            i1 = v * 16
                            i2 = (v + nb) * 16
                            ak = sk[pl.ds(i1, 16)]
                            bk = sk[pl.ds(i2, 16)]
                            ai = si[pl.ds(i1, 16)]
                            bi = si[pl.ds(i2, 16)]
                            dirn = (i1 & ks) == 0
                            cbe = (ak > bk) | ((ak == bk) & (ai < bi))
                            cond = cbe == dirn
                            sk[pl.ds(i1, 16)] = jnp.where(cond, ak, bk)
                            sk[pl.ds(i2, 16)] = jnp.where(cond, bk, ak)
                            si[pl.ds(i1, 16)] = jnp.where(cond, ai, bi)
                            si[pl.ds(i2, 16)] = jnp.where(cond, bi, ai)
                    else:
                        @plsc.parallel_loop(0, CAP2 // 16, 1, unroll=4)
                        def inner(v, ks=ks, jj=jj):
                            base = v * 16
                            ak = sk[pl.ds(base, 16)]
                            ai = si[pl.ds(base, 16)]
                            pidx = base + (iota ^ jj)
                            bk = plsc.load_gather(sk, [pidx])
                            bi = plsc.load_gather(si, [pidx])
                            dirv = ((base + iota) & ks) == 0
                            keepf = (iota & jj) == 0
                            cbe = (ak > bk) | ((ak == bk) & (ai < bi))
                            cond = (cbe == dirv) == keepf
                            sk[pl.ds(base, 16)] = jnp.where(cond, ak, bk)
                            si[pl.ds(base, 16)] = jnp.where(cond, ai, bi)
                    jj >>= 1

            @plsc.parallel_loop(0, KPAD, 16, unroll=2)
            def p5(t):
                k = sk[pl.ds(t, 16)]
                ix = si[pl.ds(t, 16)]
                v = lax.bitcast_convert_type(
                    k ^ ((k >> 31) & 0x7FFFFFFF), jnp.float32)
                sc = 1.0 / (1.0 + jnp.exp(-v))
                q = lax.div(ix, C)
                olab[pl.ds(t, 16)] = ix - q * C
                osc[pl.ds(t, 16)] = sc
                qb[pl.ds(t, 16)] = q

            scvv = scv[...]

            @plsc.parallel_loop(0, KPAD * 4, 16, unroll=4)
            def p6(t):
                qv = plsc.load_gather(qb, [(t >> 2) + qsh])
                g = plsc.load_gather(bxv, [qv * 4 + c3])
                p = plsc.load_gather(bxv, [qv * 4 + c3x2])
                res = jnp.where(lo_mask, g - 0.5 * p, p + 0.5 * g)
                obx[pl.ds(t, 16)] = res * scvv

            pltpu.sync_copy(olab, lab_hbm.at[r])
            pltpu.sync_copy(obx, box_hbm.at[r])
            pltpu.sync_copy(osc, sc_hbm.at[r])
            return 0

        lax.fori_loop(0, ROWS_PER_WORKER, row_body, 0)

    return sc_call


_sc_call = _make_sc_call()


def kernel(pred_logits, pred_boxes, orig_target_sizes):
    logits2d = pred_logits.reshape(BATCH, N)
    boxes2d = pred_boxes.reshape(BATCH, Q * 4)
    scale16 = jnp.tile(orig_target_sizes, (1, 8))
    lab_p, box_p, sc_p = _sc_call(logits2d, boxes2d, scale16)
    labels = lab_p[:, :K]
    boxes = box_p.reshape(BATCH, KPAD, 4)[:, :K]
    scores = sc_p[:, :K]
    return (labels, boxes, scores)


# speedup vs baseline: 1.6432x; 1.0785x over previous
import functools

import jax
import jax.numpy as jnp
from jax import lax
from jax.experimental import pallas as pl
from jax.experimental.pallas import tpu as pltpu
from jax.experimental.pallas import tpu_sc as plsc

BATCH = 128
Q = 1000
C = 80
N = Q * C
K = 300
KPAD = 320
NB1 = 1024
NB2 = 256
CAP1 = 4096
CAP2 = 512
ROWS_PER_WORKER = BATCH // 32
INT_MIN = -2147483648


def _mono_key(v):
    b = lax.bitcast_convert_type(v, jnp.int32)
    return b ^ ((b >> 31) & 0x7FFFFFFF)


def _make_sc_call():
    mesh = plsc.VectorSubcoreMesh(core_axis_name="c", subcore_axis_name="s")

    @functools.partial(
        pl.kernel,
        mesh=mesh,
        compiler_params=pltpu.CompilerParams(needs_layout_passes=False),
        out_type=[
            jax.ShapeDtypeStruct((BATCH, KPAD), jnp.int32),
            jax.ShapeDtypeStruct((BATCH, KPAD * 4), jnp.float32),
            jax.ShapeDtypeStruct((BATCH, KPAD), jnp.float32),
        ],
        scratch_types=[
            pltpu.VMEM((N,), jnp.float32),
            pltpu.VMEM((NB1 * 16,), jnp.int32),
            pltpu.VMEM((CAP1,), jnp.int32),
            pltpu.VMEM((CAP1,), jnp.int32),
            pltpu.VMEM((CAP1,), jnp.int32),
            pltpu.VMEM((NB2 * 16,), jnp.int32),
            pltpu.VMEM((CAP2,), jnp.int32),
            pltpu.VMEM((CAP2,), jnp.int32),
            pltpu.VMEM((KPAD,), jnp.int32),
            pltpu.VMEM((Q * 4,), jnp.float32),
            pltpu.VMEM((16,), jnp.float32),
            pltpu.VMEM((KPAD,), jnp.int32),
            pltpu.VMEM((KPAD * 4,), jnp.float32),
            pltpu.VMEM((KPAD,), jnp.float32),
        ],
    )
    def sc_call(logits_hbm, boxes_hbm, scale_hbm,
                lab_hbm, box_hbm, sc_hbm,
                xb, h1, ck, ci, ci2, h2, sk, si, qb, bxv, scv,
                olab, obx, osc):
        wid = lax.axis_index("s") * 2 + lax.axis_index("c")
        iota = lax.iota(jnp.int32, 16)
        ones = jnp.ones((16,), jnp.int32)
        zero16 = jnp.zeros((16,), jnp.int32)
        sentk = jnp.full((16,), INT_MIN, jnp.int32)
        qsh = iota >> 2
        c3 = iota & 3
        c3x2 = c3 ^ 2
        lo_mask = c3 < 2
        h1base = 8192 + iota
        h2base = iota

        def row_body(j, _):
            r = wid * ROWS_PER_WORKER + j
            pltpu.sync_copy(logits_hbm.at[r], xb)
            pltpu.sync_copy(boxes_hbm.at[r], bxv)
            pltpu.sync_copy(scale_hbm.at[r], scv)

            @plsc.parallel_loop(0, NB1 * 16, 16, unroll=8)
            def z1(i):
                h1[pl.ds(i, 16)] = zero16

            @plsc.parallel_loop(0, NB2 * 16, 16, unroll=8)
            def z2(i):
                h2[pl.ds(i, 16)] = zero16

            @plsc.parallel_loop(0, N, 16, unroll=4)
            def p1(i):
                k = _mono_key(xb[pl.ds(i, 16)])
                idx = ((k >> 18) & -16) + h1base
                plsc.addupdate_scatter(h1, [idx], ones)

            def sfxg(t, cc):
                carry, cnt, aboveg = cc
                base = (63 - t) * 256
                tot = h1[pl.ds(base, 16)]
                for l in range(1, 16):
                    tot = tot + h1[pl.ds(base + l * 16, 16)]
                incl = carry + jnp.sum(tot)
                hit = incl >= K
                first = hit & (cnt == 0)
                aboveg = jnp.where(first, carry, aboveg)
                return (incl, cnt + jnp.where(hit, 1, 0), aboveg)

            _, cntg, aboveg = lax.fori_loop(
                0, NB1 // 16, sfxg, (jnp.int32(0), jnp.int32(0), jnp.int32(0)))
            gg = cntg - 1

            btot = zero16
            for l in range(16):
                btot = btot + plsc.load_gather(h1, [gg * 256 + iota * 16 + l])
            rsuf = lax.rev(plsc.cumsum(lax.rev(btot, (0,))), (0,))
            inclb = aboveg + rsuf
            cntb = jnp.sum(jnp.where(inclb >= K, 1, 0))
            lsel = cntb - 1
            bb = gg * 16 + lsel
            above_s = jnp.sum(jnp.where(iota == lsel, inclb - btot, 0))
            t1 = (bb - 512) * (1 << 22)
            t1splat = jnp.broadcast_to(t1, (16,))

            @plsc.parallel_loop(0, N, 16, unroll=4, carry=zero16)
            def p2(i, pc):
                k = _mono_key(xb[pl.ds(i, 16)])
                m = k >= t1splat
                pos = jnp.minimum(pc + iota, CAP1 - 16 + iota)
                plsc.store_scatter(ci2, [pos], i + iota, mask=m)
                return pc + jnp.where(m, 16, 0)

            pc16 = p2
            cvn = jnp.minimum(jnp.max(pc16), CAP1)

            @plsc.parallel_loop(0, cvn, 16, unroll=2, carry=zero16)
            def p25(cpos, off):
                m = pc16 > cpos
                ixv = ci2[pl.ds(cpos, 16)]
                xv = plsc.load_gather(xb, [ixv], mask=m)
                k = _mono_key(xv)
                cs = plsc.cumsum(jnp.where(m, 1, 0))
                pos = jnp.maximum(jnp.minimum(off + cs - 1, CAP1 - 1), 0)
                plsc.store_scatter(ck, [pos], k, mask=m)
                plsc.store_scatter(ci, [pos], ixv, mask=m)
                return off + plsc.all_reduce_population_count(m)

            n_cand = jnp.minimum(jnp.max(p25), CAP1)
            ncand16 = ((n_cand + 15) // 16) * 16

            t1hi = t1 + (1 << 22)
            t1hisplat = jnp.broadcast_to(t1hi, (16,))

            @plsc.parallel_loop(0, ncand16, 16, unroll=2)
            def p3(i):
                k = ck[pl.ds(i, 16)]
                m = (k >= t1splat) & (k < t1hisplat) & ((i + iota) < n_cand)
                idx = ((k >> 10) & 0xFF0) + h2base
                plsc.addupdate_scatter(h2, [idx], ones, mask=m)

            def sfx2(t, cc):
                carry, cnt, aboveg2 = cc
                base = (15 - t) * 256
                tot = h2[pl.ds(base, 16)]
                for l in range(1, 16):
                    tot = tot + h2[pl.ds(base + l * 16, 16)]
                incl = carry + jnp.sum(tot)
                hit = (above_s + incl) >= K
                first = hit & (cnt == 0)
                aboveg2 = jnp.where(first, carry, aboveg2)
                return (incl, cnt + jnp.where(hit, 1, 0), aboveg2)

            _, cntg2, aboveg2 = lax.fori_loop(
                0, NB2 // 16, sfx2, (jnp.int32(0), jnp.int32(0), jnp.int32(0)))
            gg2 = cntg2 - 1
            btot2 = zero16
            for l in range(16):
                btot2 = btot2 + plsc.load_gather(
                    h2, [gg2 * 256 + iota * 16 + l])
            rsuf2 = lax.rev(plsc.cumsum(lax.rev(btot2, (0,))), (0,))
            inclb2 = above_s + aboveg2 + rsuf2
            cntb2 = jnp.sum(jnp.where(inclb2 >= K, 1, 0))
            dd2 = gg2 * 16 + (cntb2 - 1)
            t2 = t1 + dd2 * (1 << 14)
            t2splat = jnp.broadcast_to(t2, (16,))

            @plsc.parallel_loop(0, CAP2, 16, unroll=4)
            def zs(i):
                sk[pl.ds(i, 16)] = sentk
                si[pl.ds(i, 16)] = zero16

            @plsc.parallel_loop(0, ncand16, 16, unroll=2, carry=zero16)
            def p4(i, off):
                k = ck[pl.ds(i, 16)]
                m = (k >= t2splat) & ((i + iota) < n_cand)
                cs = plsc.cumsum(jnp.where(m, 1, 0))
                pos = jnp.maximum(jnp.minimum(off + cs - 1, CAP2 - 1), 0)
                plsc.store_scatter(sk, [pos], k, mask=m)
                plsc.store_scatter(si, [pos], ci[pl.ds(i, 16)], mask=m)
                return off + plsc.all_reduce_population_count(m)

            _ = p4

            for ks in [2 << s for s in range(9)]:
                jj = ks >> 1
                while jj >= 1:
                    if jj >= 16:
                        nb = jj // 16
                        lnb = nb.bit_length() - 1

                        @plsc.parallel_loop(0, CAP2 // 32, 1, unroll=4)
                        def cross(t, ks=ks, nb=nb, lnb=lnb):
                            v = ((t >> lnb) << (lnb + 1)) + (t & (nb - 1))
                            i1 = v * 16
                            i2 = (v + nb) * 16
                            ak = sk[pl.ds(i1, 16)]
                            bk = sk[pl.ds(i2, 16)]
                            ai = si[pl.ds(i1, 16)]
                            bi = si[pl.ds(i2, 16)]
                            dirn = (i1 & ks) == 0
                            cbe = (ak > bk) | ((ak == bk) & (ai < bi))
                            cond = cbe == dirn
                            sk[pl.ds(i1, 16)] = jnp.where(cond, ak, bk)
                            sk[pl.ds(i2, 16)] = jnp.where(cond, bk, ak)
                            si[pl.ds(i1, 16)] = jnp.where(cond, ai, bi)
                            si[pl.ds(i2, 16)] = jnp.where(cond, bi, ai)
                    else:
                        @plsc.parallel_loop(0, CAP2 // 16, 1, unroll=4)
                        def inner(v, ks=ks, jj=jj):
                            base = v * 16
                            ak = sk[pl.ds(base, 16)]
                            ai = si[pl.ds(base, 16)]
                            pidx = base + (iota ^ jj)
                            bk = plsc.load_gather(sk, [pidx])
                            bi = plsc.load_gather(si, [pidx])
                            dirv = ((base + iota) & ks) == 0
                            keepf = (iota & jj) == 0
                            cbe = (ak > bk) | ((ak == bk) & (ai < bi))
                            cond = (cbe == dirv) == keepf
                            sk[pl.ds(base, 16)] = jnp.where(cond, ak, bk)
                            si[pl.ds(base, 16)] = jnp.where(cond, ai, bi)
                    jj >>= 1

            @plsc.parallel_loop(0, KPAD, 16, unroll=2)
            def p5(t):
                k = sk[pl.ds(t, 16)]
                ix = si[pl.ds(t, 16)]
                v = lax.bitcast_convert_type(
                    k ^ ((k >> 31) & 0x7FFFFFFF), jnp.float32)
                sc = 1.0 / (1.0 + jnp.exp(-v))
                q = lax.div(ix, C)
                olab[pl.ds(t, 16)] = ix - q * C
                osc[pl.ds(t, 16)] = sc
                qb[pl.ds(t, 16)] = q

            scvv = scv[...]

            @plsc.parallel_loop(0, KPAD * 4, 16, unroll=4)
            def p6(t):
                qv = plsc.load_gather(qb, [(t >> 2) + qsh])
                g = plsc.load_gather(bxv, [qv * 4 + c3])
                p = plsc.load_gather(bxv, [qv * 4 + c3x2])
                res = jnp.where(lo_mask, g - 0.5 * p, p + 0.5 * g)
                obx[pl.ds(t, 16)] = res * scvv

            pltpu.sync_copy(olab, lab_hbm.at[r])
            pltpu.sync_copy(obx, box_hbm.at[r])
            pltpu.sync_copy(osc, sc_hbm.at[r])
            return 0

        lax.fori_loop(0, ROWS_PER_WORKER, row_body, 0)

    return sc_call


_sc_call = _make_sc_call()


def kernel(pred_logits, pred_boxes, orig_target_sizes):
    logits2d = pred_logits.reshape(BATCH, N)
    boxes2d = pred_boxes.reshape(BATCH, Q * 4)
    scale16 = jnp.tile(orig_target_sizes, (1, 8))
    lab_p, box_p, sc_p = _sc_call(logits2d, boxes2d, scale16)
    labels = lab_p[:, :K]
    boxes = box_p.reshape(BATCH, KPAD, 4)[:, :K]
    scores = sc_p[:, :K]
    return (labels, boxes, scores)


# speedup vs baseline: 1.6553x; 1.0074x over previous
import functools

import jax
import jax.numpy as jnp
from jax import lax
from jax.experimental import pallas as pl
from jax.experimental.pallas import tpu as pltpu
from jax.experimental.pallas import tpu_sc as plsc

BATCH = 128
Q = 1000
C = 80
N = Q * C
K = 300
KPAD = 320
NB1 = 1024
NB2 = 256
CAP1 = 4096
CAP2 = 512
ROWS_PER_WORKER = BATCH // 32
INT_MIN = -2147483648


def _mono_key(v):
    b = lax.bitcast_convert_type(v, jnp.int32)
    return b ^ ((b >> 31) & 0x7FFFFFFF)


def _make_sc_call():
    mesh = plsc.VectorSubcoreMesh(core_axis_name="c", subcore_axis_name="s")

    @functools.partial(
        pl.kernel,
        mesh=mesh,
        compiler_params=pltpu.CompilerParams(needs_layout_passes=False),
        out_type=[
            jax.ShapeDtypeStruct((BATCH, KPAD), jnp.int32),
            jax.ShapeDtypeStruct((BATCH, KPAD * 4), jnp.float32),
            jax.ShapeDtypeStruct((BATCH, KPAD), jnp.float32),
        ],
        scratch_types=[
            pltpu.VMEM((N,), jnp.float32),
            pltpu.VMEM((NB1 * 16,), jnp.int32),
            pltpu.VMEM((CAP1,), jnp.int32),
            pltpu.VMEM((CAP1,), jnp.int32),
            pltpu.VMEM((CAP1,), jnp.int32),
            pltpu.VMEM((NB2 * 16,), jnp.int32),
            pltpu.VMEM((CAP2,), jnp.int32),
            pltpu.VMEM((CAP2,), jnp.int32),
            pltpu.VMEM((KPAD,), jnp.int32),
            pltpu.VMEM((Q * 4,), jnp.float32),
            pltpu.VMEM((16,), jnp.float32),
            pltpu.VMEM((KPAD,), jnp.int32),
            pltpu.VMEM((KPAD * 4,), jnp.float32),
            pltpu.VMEM((KPAD,), jnp.float32),
        ],
    )
    def sc_call(logits_hbm, boxes_hbm, scale_hbm,
                lab_hbm, box_hbm, sc_hbm,
                xb, h1, ck, ci, ci2, h2, sk, si, qb, bxv, scv,
                olab, obx, osc):
        wid = lax.axis_index("s") * 2 + lax.axis_index("c")
        iota = lax.iota(jnp.int32, 16)
        ones = jnp.ones((16,), jnp.int32)
        zero16 = jnp.zeros((16,), jnp.int32)
        sentk = jnp.full((16,), INT_MIN, jnp.int32)
        qsh = iota >> 2
        c3 = iota & 3
        c3x2 = c3 ^ 2
        lo_mask = c3 < 2
        h1base = 8192 + iota
        h2base = iota

        def row_body(j, _):
            r = wid * ROWS_PER_WORKER + j
            pltpu.sync_copy(logits_hbm.at[r], xb)
            pltpu.sync_copy(boxes_hbm.at[r], bxv)
            pltpu.sync_copy(scale_hbm.at[r], scv)

            @plsc.parallel_loop(0, NB1 * 16, 16, unroll=8)
            def z1(i):
                h1[pl.ds(i, 16)] = zero16

            @plsc.parallel_loop(0, NB2 * 16, 16, unroll=8)
            def z2(i):
                h2[pl.ds(i, 16)] = zero16

            @plsc.parallel_loop(0, N, 16, unroll=8)
            def p1(i):
                k = _mono_key(xb[pl.ds(i, 16)])
                idx = ((k >> 18) & -16) + h1base
                plsc.addupdate_scatter(h1, [idx], ones)

            def sfxg(t, cc):
                carry, cnt, aboveg = cc
                base = (63 - t) * 256
                tot = h1[pl.ds(base, 16)]
                for l in range(1, 16):
                    tot = tot + h1[pl.ds(base + l * 16, 16)]
                incl = carry + jnp.sum(tot)
                hit = incl >= K
                first = hit & (cnt == 0)
                aboveg = jnp.where(first, carry, aboveg)
                return (incl, cnt + jnp.where(hit, 1, 0), aboveg)

            _, cntg, aboveg = lax.fori_loop(
                0, NB1 // 16, sfxg, (jnp.int32(0), jnp.int32(0), jnp.int32(0)))
            gg = cntg - 1

            btot = zero16
            for l in range(16):
                btot = btot + plsc.load_gather(h1, [gg * 256 + iota * 16 + l])
            rsuf = lax.rev(plsc.cumsum(lax.rev(btot, (0,))), (0,))
            inclb = aboveg + rsuf
            cntb = jnp.sum(jnp.where(inclb >= K, 1, 0))
            lsel = cntb - 1
            bb = gg * 16 + lsel
            above_s = jnp.sum(jnp.where(iota == lsel, inclb - btot, 0))
            t1 = (bb - 512) * (1 << 22)
            t1splat = jnp.broadcast_to(t1, (16,))

            @plsc.parallel_loop(0, N, 16, unroll=8, carry=zero16)
            def p2(i, pc):
                k = _mono_key(xb[pl.ds(i, 16)])
                m = k >= t1splat
                pos = jnp.minimum(pc + iota, CAP1 - 16 + iota)
                plsc.store_scatter(ci2, [pos], i + iota, mask=m)
                return pc + jnp.where(m, 16, 0)

            pc16 = p2
            cvn = jnp.minimum(jnp.max(pc16), CAP1)

            @plsc.parallel_loop(0, cvn, 16, unroll=2, carry=zero16)
            def p25(cpos, off):
                m = pc16 > cpos
                ixv = ci2[pl.ds(cpos, 16)]
                xv = plsc.load_gather(xb, [ixv], mask=m)
                k = _mono_key(xv)
                cs = plsc.cumsum(jnp.where(m, 1, 0))
                pos = jnp.maximum(jnp.minimum(off + cs - 1, CAP1 - 1), 0)
                plsc.store_scatter(ck, [pos], k, mask=m)
                plsc.store_scatter(ci, [pos], ixv, mask=m)
                return off + plsc.all_reduce_population_count(m)

            n_cand = jnp.minimum(jnp.max(p25), CAP1)
            ncand16 = ((n_cand + 15) // 16) * 16

            t1hi = t1 + (1 << 22)
            t1hisplat = jnp.broadcast_to(t1hi, (16,))

            @plsc.parallel_loop(0, ncand16, 16, unroll=2)
            def p3(i):
                k = ck[pl.ds(i, 16)]
                m = (k >= t1splat) & (k < t1hisplat) & ((i + iota) < n_cand)
                idx = ((k >> 10) & 0xFF0) + h2base
                plsc.addupdate_scatter(h2, [idx], ones, mask=m)

            def sfx2(t, cc):
                carry, cnt, aboveg2 = cc
                base = (15 - t) * 256
                tot = h2[pl.ds(base, 16)]
                for l in range(1, 16):
                    tot = tot + h2[pl.ds(base + l * 16, 16)]
                incl = carry + jnp.sum(tot)
                hit = (above_s + incl) >= K
                first = hit & (cnt == 0)
                aboveg2 = jnp.where(first, carry, aboveg2)
                return (incl, cnt + jnp.where(hit, 1, 0), aboveg2)

            _, cntg2, aboveg2 = lax.fori_loop(
                0, NB2 // 16, sfx2, (jnp.int32(0), jnp.int32(0), jnp.int32(0)))
            gg2 = cntg2 - 1
            btot2 = zero16
            for l in range(16):
                btot2 = btot2 + plsc.load_gather(
                    h2, [gg2 * 256 + iota * 16 + l])
            rsuf2 = lax.rev(plsc.cumsum(lax.rev(btot2, (0,))), (0,))
            inclb2 = above_s + aboveg2 + rsuf2
            cntb2 = jnp.sum(jnp.where(inclb2 >= K, 1, 0))
            dd2 = gg2 * 16 + (cntb2 - 1)
            t2 = t1 + dd2 * (1 << 14)
            t2splat = jnp.broadcast_to(t2, (16,))

            @plsc.parallel_loop(0, CAP2, 16, unroll=4)
            def zs(i):
                sk[pl.ds(i, 16)] = sentk
                si[pl.ds(i, 16)] = zero16

            @plsc.parallel_loop(0, ncand16, 16, unroll=2, carry=zero16)
            def p4(i, off):
                k = ck[pl.ds(i, 16)]
                m = (k >= t2splat) & ((i + iota) < n_cand)
                cs = plsc.cumsum(jnp.where(m, 1, 0))
                pos = jnp.maximum(jnp.minimum(off + cs - 1, CAP2 - 1), 0)
                plsc.store_scatter(sk, [pos], k, mask=m)
                plsc.store_scatter(si, [pos], ci[pl.ds(i, 16)], mask=m)
                return off + plsc.all_reduce_population_count(m)

            _ = p4

            for ks in [2 << s for s in range(9)]:
                jj = ks >> 1
                while jj >= 1:
                    if jj >= 16:
                        nb = jj // 16
                        lnb = nb.bit_length() - 1

                        @plsc.parallel_loop(0, CAP2 // 32, 1, unroll=4)
                        def cross(t, ks=ks, nb=nb, lnb=lnb):
                            v = ((t >> lnb) << (lnb + 1)) + (t & (nb - 1))
                            i1 = v * 16
                            i2 = (v + nb) * 16
                            ak = sk[pl.ds(i1, 16)]
                            bk = sk[pl.ds(i2, 16)]
                            ai = si[pl.ds(i1, 16)]
                            bi = si[pl.ds(i2, 16)]
                            dirn = (i1 & ks) == 0
                            cbe = (ak > bk) | ((ak == bk) & (ai < bi))
                            cond = cbe == dirn
                            sk[pl.ds(i1, 16)] = jnp.where(cond, ak, bk)
                            sk[pl.ds(i2, 16)] = jnp.where(cond, bk, ak)
                            si[pl.ds(i1, 16)] = jnp.where(cond, ai, bi)
                            si[pl.ds(i2, 16)] = jnp.where(cond, bi, ai)
                    else:
                        @plsc.parallel_loop(0, CAP2 // 16, 1, unroll=4)
                        def inner(v, ks=ks, jj=jj):
                            base = v * 16
                            ak = sk[pl.ds(base, 16)]
                            ai = si[pl.ds(base, 16)]
                            pidx = base + (iota ^ jj)
                            bk = plsc.load_gather(sk, [pidx])
                            bi = plsc.load_gather(si, [pidx])
                            dirv = ((base + iota) & ks) == 0
                            keepf = (iota & jj) == 0
                            cbe = (ak > bk) | ((ak == bk) & (ai < bi))
                            cond = (cbe == dirv) == keepf
                            sk[pl.ds(base, 16)] = jnp.where(cond, ak, bk)
                            si[pl.ds(base, 16)] = jnp.where(cond, ai, bi)
                    jj >>= 1

            @plsc.parallel_loop(0, KPAD, 16, unroll=2)
            def p5(t):
                k = sk[pl.ds(t, 16)]
                ix = si[pl.ds(t, 16)]
                v = lax.bitcast_convert_type(
                    k ^ ((k >> 31) & 0x7FFFFFFF), jnp.float32)
                sc = 1.0 / (1.0 + jnp.exp(-v))
                q = lax.div(ix, C)
                olab[pl.ds(t, 16)] = ix - q * C
                osc[pl.ds(t, 16)] = sc
                qb[pl.ds(t, 16)] = q

            scvv = scv[...]

            @plsc.parallel_loop(0, KPAD * 4, 16, unroll=4)
            def p6(t):
                qv = plsc.load_gather(qb, [(t >> 2) + qsh])
                g = plsc.load_gather(bxv, [qv * 4 + c3])
                p = plsc.load_gather(bxv, [qv * 4 + c3x2])
                res = jnp.where(lo_mask, g - 0.5 * p, p + 0.5 * g)
                obx[pl.ds(t, 16)] = res * scvv

            pltpu.sync_copy(olab, lab_hbm.at[r])
            pltpu.sync_copy(obx, box_hbm.at[r])
            pltpu.sync_copy(osc, sc_hbm.at[r])
            return 0

        lax.fori_loop(0, ROWS_PER_WORKER, row_body, 0)

    return sc_call


_sc_call = _make_sc_call()


def kernel(pred_logits, pred_boxes, orig_target_sizes):
    logits2d = pred_logits.reshape(BATCH, N)
    boxes2d = pred_boxes.reshape(BATCH, Q * 4)
    scale16 = jnp.tile(orig_target_sizes, (1, 8))
    lab_p, box_p, sc_p = _sc_call(logits2d, boxes2d, scale16)
    labels = lab_p[:, :K]
    boxes = box_p.reshape(BATCH, KPAD, 4)[:, :K]
    scores = sc_p[:, :K]
    return (labels, boxes, scores)
